# baseline (device time: 86706 ns/iter reference)
import functools

import jax
import jax.numpy as jnp
from jax import lax
from jax.experimental import pallas as pl
from jax.experimental.pallas import tpu as pltpu

N_DEV = 8
B, SQ, D = 4, 256, 1024
HQ, HKV, DH = 8, 2, 128
SCALE = 0.08838834764831843
R = B * SQ
BLK = 1024 // N_DEV
SBLK = SQ // N_DEV
HALF = SQ // 2


def _combine(my_o, my_m, my_l, in_o, in_m, in_l, nblk):
    m_n = jnp.maximum(my_m, in_m)
    a_my = jnp.exp(my_m - m_n)
    a_in = jnp.exp(in_m - m_n)
    l_n = my_l * a_my + in_l * a_in
    o_n = (
        my_o * a_my.reshape(nblk, B, HKV, BLK, 1)
        + in_o * a_in.reshape(nblk, B, HKV, BLK, 1)
    )
    return o_n, m_n, l_n


def _fused_body(
    x_ref, wq_ref, wo_ref, k_ref, v_ref, out_ref,
    q_ref, sd_o, sd_ml, kp_o, kp_ml,
    px_o, px_ml, rx_o, rx_ml,
    yx,
    o_send, o_recv, ml_send, ml_recv, ag_send, ag_recv,
):
    me = lax.axis_index("i")
    partners = [
        lax.bitwise_xor(me, 4),
        lax.bitwise_xor(me, 2),
        lax.bitwise_xor(me, 1),
    ]

    barrier_sem = pltpu.get_barrier_semaphore()
    for p in partners:
        pl.semaphore_signal(
            barrier_sem, inc=1,
            device_id=(p,), device_id_type=pl.DeviceIdType.MESH,
        )
    pl.semaphore_wait(barrier_sem, 3)

    X = x_ref[...].reshape(R, D).astype(jnp.bfloat16)
    Wq = wq_ref[...].astype(jnp.bfloat16)
    Q = jnp.dot(X, Wq, preferred_element_type=jnp.float32)
    q_ref[...] = (Q * SCALE).astype(jnp.bfloat16).reshape(B, SQ, HQ, DH)

    def compute_half(base, o_dst, ml_dst):
        for b in range(B):
            Qb = q_ref[b, pl.ds(base * SBLK, HALF)]
            for g in range(HKV):
                Qg = Qb[:, 4 * g : 4 * g + 4, :].reshape(HALF * 4, DH)
                Kg = k_ref[b, :, g, :].astype(jnp.bfloat16)
                Vg = v_ref[b, :, g, :].astype(jnp.bfloat16)
                s = lax.dot_general(
                    Qg, Kg, (((1,), (1,)), ((), ())),
                    preferred_element_type=jnp.float32,
                )
                mx = jnp.max(s, axis=1)
                p = jnp.exp(s - mx[:, None])
                ls = jnp.sum(p, axis=1)
                o = jnp.dot(
                    p.astype(jnp.bfloat16), Vg,
                    preferred_element_type=jnp.float32,
                ).astype(jnp.bfloat16)
                o_dst[:, b, g] = o.reshape(4, BLK, DH)
                ml_dst[:, 0, 2 * b + g] = mx.reshape(4, BLK)
                ml_dst[:, 1, 2 * b + g] = ls.reshape(4, BLK)

    kb0 = lax.bitwise_and(me, 4)
    sb0 = lax.bitwise_xor(kb0, 4)
    compute_half(sb0, sd_o, sd_ml)
    compute_half(kb0, kp_o, kp_ml)
    for m in range(N_DEV):
        lm = lax.bitwise_and(lax.bitwise_xor(me, m), 3)
        ho, hml = (kp_o, kp_ml) if m < 4 else (sd_o, sd_ml)
        px_o[m] = ho[pl.ds(lm, 1)][0]
        px_ml[m] = hml[pl.ds(lm, 1)][0]

    def px_exch(src_m, k, partner):
        ro = pltpu.make_async_remote_copy(
            src_ref=px_o.at[src_m], dst_ref=rx_o.at[k],
            send_sem=o_send.at[k], recv_sem=o_recv.at[k],
            device_id=(partner,), device_id_type=pl.DeviceIdType.MESH,
        )
        rm = pltpu.make_async_remote_copy(
            src_ref=px_ml.at[src_m], dst_ref=rx_ml.at[k],
            send_sem=ml_send.at[k], recv_sem=ml_recv.at[k],
            device_id=(partner,), device_id_type=pl.DeviceIdType.MESH,
        )
        ro.start()
        rm.start()
        return ro, rm

    def px_wait(h):
        h[0].wait()
        h[1].wait()

    def _cmb1(o_a, m_a, l_a, o_b, m_b, l_b):
        m_n = jnp.maximum(m_a, m_b)
        a_a = jnp.exp(m_a - m_n)
        a_b = jnp.exp(m_b - m_n)
        l_n = l_a * a_a + l_b * a_b
        o_n = (
            o_a * a_a.reshape(B, HKV, BLK, 1)
            + o_b * a_b.reshape(B, HKV, BLK, 1)
        )
        return o_n, m_n, l_n

    def rx_combine(m, k):
        o_n, m_n, l_n = _cmb1(
            px_o[m].astype(jnp.float32), px_ml[m, 0], px_ml[m, 1],
            rx_o[k].astype(jnp.float32), rx_ml[k, 0], rx_ml[k, 1],
        )
        px_o[m] = o_n.astype(jnp.bfloat16)
        px_ml[m, 0] = m_n
        px_ml[m, 1] = l_n

    h70 = px_exch(7, 0, partners[2])
    h31 = px_exch(3, 1, partners[2])
    h52 = px_exch(5, 2, partners[0])
    px_wait(h70)
    rx_combine(6, 0)
    h63 = px_exch(6, 3, partners[1])
    px_wait(h31)
    rx_combine(2, 1)
    px_wait(h52)
    rx_combine(1, 2)
    px_wait(h63)
    rx_combine(4, 3)
    h14 = px_exch(1, 4, partners[2])
    h25 = px_exch(2, 5, partners[1])
    h46 = px_exch(4, 6, partners[0])
    px_wait(h14)
    px_wait(h25)
    px_wait(h46)
    o_n, m_n, l_n = _cmb1(
        px_o[0].astype(jnp.float32), px_ml[0, 0], px_ml[0, 1],
        rx_o[4].astype(jnp.float32), rx_ml[4, 0], rx_ml[4, 1],
    )
    o_n, m_n, l_n = _cmb1(
        o_n, m_n, l_n,
        rx_o[5].astype(jnp.float32), rx_ml[5, 0], rx_ml[5, 1],
    )
    o_n, m_n, l_n = _cmb1(
        o_n, m_n, l_n,
        rx_o[6].astype(jnp.float32), rx_ml[6, 0], rx_ml[6, 1],
    )
    o_n = o_n[None]
    l_n = l_n[None]

    Wo = wo_ref[...].astype(jnp.bfloat16)
    o_f = (o_n[0] / l_n[0].reshape(B, HKV, BLK, 1)).astype(jnp.bfloat16)
    mat = jnp.stack(
        [
            jnp.concatenate(
                [o_f[b, g].reshape(SBLK, 4 * DH) for g in range(HKV)],
                axis=1,
            )
            for b in range(B)
        ]
    ).reshape(B * SBLK, D)
    y = jnp.dot(mat, Wo, preferred_element_type=jnp.float32)
    yx[0] = y.astype(jnp.bfloat16)

    def y_exch(src_m, dst_m, partner, k):
        r = pltpu.make_async_remote_copy(
            src_ref=yx.at[src_m], dst_ref=yx.at[dst_m],
            send_sem=ag_send.at[k], recv_sem=ag_recv.at[k],
            device_id=(partner,), device_id_type=pl.DeviceIdType.MESH,
        )
        r.start()
        return r

    def y_store(m):
        bm = lax.bitwise_xor(me, m)
        out_ref[:, pl.ds(bm * SBLK, SBLK), :] = yx[m].reshape(B, SBLK, D)

    p1 = [
        y_exch(0, 1, partners[2], 0),
        y_exch(0, 2, partners[1], 1),
        y_exch(0, 4, partners[0], 2),
    ]
    out_ref[:, pl.ds(me * SBLK, SBLK), :] = (
        y.astype(jnp.bfloat16).reshape(B, SBLK, D)
    )
    for r in p1:
        r.wait()
    p2 = [
        y_exch(2, 3, partners[2], 3),
        y_exch(4, 6, partners[1], 4),
        y_exch(1, 5, partners[0], 5),
    ]
    y_store(1)
    y_store(2)
    y_store(4)
    for r in p2:
        r.wait()
    p3 = [y_exch(6, 7, partners[2], 6)]
    y_store(3)
    y_store(6)
    y_store(5)
    for r in p3:
        r.wait()
    y_store(7)

    @functools.partial(
        pl.run_scoped, second_barrier=pltpu.SemaphoreType.REGULAR
    )
    def _(second_barrier):
        for p in partners:
            pl.semaphore_signal(
                second_barrier, inc=1,
                device_id=(p,), device_id_type=pl.DeviceIdType.MESH,
            )
        pl.semaphore_wait(second_barrier, 3)


def kernel(x, Wq, Wo, K_ext, V_ext):
    return pl.pallas_call(
        _fused_body,
        out_shape=jax.ShapeDtypeStruct((B, SQ, D), jnp.bfloat16),
        in_specs=[pl.BlockSpec(memory_space=pltpu.VMEM)] * 5,
        out_specs=pl.BlockSpec(memory_space=pltpu.VMEM),
        scratch_shapes=[
            pltpu.VMEM((B, SQ, HQ, DH), jnp.bfloat16),
            pltpu.VMEM((4, B, HKV, BLK, DH), jnp.bfloat16),
            pltpu.VMEM((4, 2, B * HKV, BLK), jnp.float32),
            pltpu.VMEM((4, B, HKV, BLK, DH), jnp.bfloat16),
            pltpu.VMEM((4, 2, B * HKV, BLK), jnp.float32),
            pltpu.VMEM((N_DEV, B, HKV, BLK, DH), jnp.bfloat16),
            pltpu.VMEM((N_DEV, 2, B * HKV, BLK), jnp.float32),
            pltpu.VMEM((7, B, HKV, BLK, DH), jnp.bfloat16),
            pltpu.VMEM((7, 2, B * HKV, BLK), jnp.float32),
            pltpu.VMEM((N_DEV, B * SBLK, D), jnp.bfloat16),
            pltpu.SemaphoreType.DMA((7,)),
            pltpu.SemaphoreType.DMA((7,)),
            pltpu.SemaphoreType.DMA((7,)),
            pltpu.SemaphoreType.DMA((7,)),
            pltpu.SemaphoreType.DMA((7,)),
            pltpu.SemaphoreType.DMA((7,)),
        ],
        compiler_params=pltpu.CompilerParams(
            collective_id=0, vmem_limit_bytes=100 * 1024 * 1024
        ),
    )(x, Wq, Wo, K_ext, V_ext)


# device time: 83554 ns/iter; 1.0377x vs baseline; 1.0377x over previous
import functools

import jax
import jax.numpy as jnp
from jax import lax
from jax.experimental import pallas as pl
from jax.experimental.pallas import tpu as pltpu

N_DEV = 8
B, SQ, D = 4, 256, 1024
HQ, HKV, DH = 8, 2, 128
SCALE = 0.08838834764831843
R = B * SQ
BLK = 1024 // N_DEV
SBLK = SQ // N_DEV
HALF = SQ // 2


def _combine(my_o, my_m, my_l, in_o, in_m, in_l, nblk):
    m_n = jnp.maximum(my_m, in_m)
    a_my = jnp.exp(my_m - m_n)
    a_in = jnp.exp(in_m - m_n)
    l_n = my_l * a_my + in_l * a_in
    o_n = (
        my_o * a_my.reshape(nblk, B, HKV, BLK, 1)
        + in_o * a_in.reshape(nblk, B, HKV, BLK, 1)
    )
    return o_n, m_n, l_n


def _fused_body(
    x_ref, wq_ref, wo_ref, k_ref, v_ref, out_ref,
    q_ref, sd_o, sd_ml, kp_o, kp_ml,
    ro0, ro1, ro2, rml0, rml1, rml2,
    c0o, c0ml, c1o, c1ml,
    yx,
    o_send, o_recv, ml_send, ml_recv, ag_send, ag_recv,
):
    me = lax.axis_index("i")
    partners = [
        lax.bitwise_xor(me, 4),
        lax.bitwise_xor(me, 2),
        lax.bitwise_xor(me, 1),
    ]

    barrier_sem = pltpu.get_barrier_semaphore()
    for p in partners:
        pl.semaphore_signal(
            barrier_sem, inc=1,
            device_id=(p,), device_id_type=pl.DeviceIdType.MESH,
        )
    pl.semaphore_wait(barrier_sem, 3)

    X = x_ref[...].reshape(R, D).astype(jnp.bfloat16)
    Wq = wq_ref[...].astype(jnp.bfloat16)
    Q = jnp.dot(X, Wq, preferred_element_type=jnp.float32)
    q_ref[...] = (Q * SCALE).astype(jnp.bfloat16).reshape(B, SQ, HQ, DH)

    def compute_half(base, o_dst, ml_dst):
        for b in range(B):
            Qb = q_ref[b, pl.ds(base * SBLK, HALF)]
            for g in range(HKV):
                Qg = Qb[:, 4 * g : 4 * g + 4, :].reshape(HALF * 4, DH)
                Kg = k_ref[b, :, g, :].astype(jnp.bfloat16)
                Vg = v_ref[b, :, g, :].astype(jnp.bfloat16)
                s = lax.dot_general(
                    Qg, Kg, (((1,), (1,)), ((), ())),
                    preferred_element_type=jnp.float32,
                )
                mx = jnp.max(s, axis=1)
                p = jnp.exp(s - mx[:, None])
                ls = jnp.sum(p, axis=1)
                o = jnp.dot(
                    p.astype(jnp.bfloat16), Vg,
                    preferred_element_type=jnp.float32,
                ).astype(jnp.bfloat16)
                o_dst[:, b, g] = o.reshape(4, BLK, DH)
                ml_dst[:, 0, 2 * b + g] = mx.reshape(4, BLK)
                ml_dst[:, 1, 2 * b + g] = ls.reshape(4, BLK)

    kb0 = lax.bitwise_and(me, 4)
    sb0 = lax.bitwise_xor(kb0, 4)
    compute_half(sb0, sd_o, sd_ml)
    r_o = pltpu.make_async_remote_copy(
        src_ref=sd_o, dst_ref=ro0,
        send_sem=o_send.at[0], recv_sem=o_recv.at[0],
        device_id=(partners[0],), device_id_type=pl.DeviceIdType.MESH,
    )
    r_ml = pltpu.make_async_remote_copy(
        src_ref=sd_ml, dst_ref=rml0,
        send_sem=ml_send.at[0], recv_sem=ml_recv.at[0],
        device_id=(partners[0],), device_id_type=pl.DeviceIdType.MESH,
    )
    r_o.start()
    r_ml.start()
    compute_half(kb0, kp_o, kp_ml)
    r_o.wait()
    r_ml.wait()
    o_n, m_n, l_n = _combine(
        kp_o[...].astype(jnp.float32), kp_ml[:, 0], kp_ml[:, 1],
        ro0[...].astype(jnp.float32), rml0[:, 0], rml0[:, 1], 4,
    )
    c0o[...] = o_n.astype(jnp.bfloat16)
    c0ml[...] = jnp.stack([m_n, l_n], axis=1)

    off_k1 = lax.bitwise_and(me, 2)
    off_s1 = lax.bitwise_xor(off_k1, 2)
    r_o = pltpu.make_async_remote_copy(
        src_ref=c0o.at[pl.ds(off_s1, 2)], dst_ref=ro1,
        send_sem=o_send.at[1], recv_sem=o_recv.at[1],
        device_id=(partners[1],), device_id_type=pl.DeviceIdType.MESH,
    )
    r_ml = pltpu.make_async_remote_copy(
        src_ref=c0ml.at[pl.ds(off_s1, 2)], dst_ref=rml1,
        send_sem=ml_send.at[1], recv_sem=ml_recv.at[1],
        device_id=(partners[1],), device_id_type=pl.DeviceIdType.MESH,
    )
    r_o.start()
    r_ml.start()
    r_o.wait()
    r_ml.wait()
    myml = c0ml[pl.ds(off_k1, 2)]
    o_n, m_n, l_n = _combine(
        c0o[pl.ds(off_k1, 2)].astype(jnp.float32), myml[:, 0], myml[:, 1],
        ro1[...].astype(jnp.float32), rml1[:, 0], rml1[:, 1], 2,
    )
    c1o[...] = o_n.astype(jnp.bfloat16)
    c1ml[...] = jnp.stack([m_n, l_n], axis=1)

    off_k2 = lax.bitwise_and(me, 1)
    off_s2 = lax.bitwise_xor(off_k2, 1)
    r_o = pltpu.make_async_remote_copy(
        src_ref=c1o.at[pl.ds(off_s2, 1)], dst_ref=ro2,
        send_sem=o_send.at[2], recv_sem=o_recv.at[2],
        device_id=(partners[2],), device_id_type=pl.DeviceIdType.MESH,
    )
    r_ml = pltpu.make_async_remote_copy(
        src_ref=c1ml.at[pl.ds(off_s2, 1)], dst_ref=rml2,
        send_sem=ml_send.at[2], recv_sem=ml_recv.at[2],
        device_id=(partners[2],), device_id_type=pl.DeviceIdType.MESH,
    )
    r_o.start()
    r_ml.start()
    r_o.wait()
    r_ml.wait()
    myml = c1ml[pl.ds(off_k2, 1)]
    o_n, m_n, l_n = _combine(
        c1o[pl.ds(off_k2, 1)].astype(jnp.float32), myml[:, 0], myml[:, 1],
        ro2[...].astype(jnp.float32), rml2[:, 0], rml2[:, 1], 1,
    )

    Wo = wo_ref[...].astype(jnp.bfloat16)
    o_f = (o_n / l_n.reshape(1, B, HKV, BLK, 1)).astype(jnp.bfloat16)[0]
    mat = jnp.stack(
        [
            jnp.concatenate(
                [o_f[b, g].reshape(SBLK, 4 * DH) for g in range(HKV)],
                axis=1,
            )
            for b in range(B)
        ]
    ).reshape(B * SBLK, D)
    y = jnp.dot(mat, Wo, preferred_element_type=jnp.float32)
    yx[0] = y.astype(jnp.bfloat16)

    def y_exch(src_m, dst_m, partner, k):
        r = pltpu.make_async_remote_copy(
            src_ref=yx.at[src_m], dst_ref=yx.at[dst_m],
            send_sem=ag_send.at[k], recv_sem=ag_recv.at[k],
            device_id=(partner,), device_id_type=pl.DeviceIdType.MESH,
        )
        r.start()
        return r

    def y_store(m):
        bm = lax.bitwise_xor(me, m)
        out_ref[:, pl.ds(bm * SBLK, SBLK), :] = yx[m].reshape(B, SBLK, D)

    p1 = [
        y_exch(0, 1, partners[2], 0),
        y_exch(0, 2, partners[1], 1),
        y_exch(0, 4, partners[0], 2),
    ]
    out_ref[:, pl.ds(me * SBLK, SBLK), :] = (
        y.astype(jnp.bfloat16).reshape(B, SBLK, D)
    )
    for r in p1:
        r.wait()
    p2 = [
        y_exch(2, 3, partners[2], 3),
        y_exch(4, 6, partners[1], 4),
        y_exch(1, 5, partners[0], 5),
    ]
    y_store(1)
    y_store(2)
    y_store(4)
    for r in p2:
        r.wait()
    p3 = [y_exch(6, 7, partners[2], 6)]
    y_store(3)
    y_store(6)
    y_store(5)
    for r in p3:
        r.wait()
    y_store(7)

    @functools.partial(
        pl.run_scoped, second_barrier=pltpu.SemaphoreType.REGULAR
    )
    def _(second_barrier):
        for p in partners:
            pl.semaphore_signal(
                second_barrier, inc=1,
                device_id=(p,), device_id_type=pl.DeviceIdType.MESH,
            )
        pl.semaphore_wait(second_barrier, 3)


def kernel(x, Wq, Wo, K_ext, V_ext):
    return pl.pallas_call(
        _fused_body,
        out_shape=jax.ShapeDtypeStruct((B, SQ, D), jnp.bfloat16),
        in_specs=[pl.BlockSpec(memory_space=pltpu.VMEM)] * 5,
        out_specs=pl.BlockSpec(memory_space=pltpu.VMEM),
        scratch_shapes=[
            pltpu.VMEM((B, SQ, HQ, DH), jnp.bfloat16),
            pltpu.VMEM((4, B, HKV, BLK, DH), jnp.bfloat16),
            pltpu.VMEM((4, 2, B * HKV, BLK), jnp.float32),
            pltpu.VMEM((4, B, HKV, BLK, DH), jnp.bfloat16),
            pltpu.VMEM((4, 2, B * HKV, BLK), jnp.float32),
            pltpu.VMEM((4, B, HKV, BLK, DH), jnp.bfloat16),
            pltpu.VMEM((2, B, HKV, BLK, DH), jnp.bfloat16),
            pltpu.VMEM((1, B, HKV, BLK, DH), jnp.bfloat16),
            pltpu.VMEM((4, 2, B * HKV, BLK), jnp.float32),
            pltpu.VMEM((2, 2, B * HKV, BLK), jnp.float32),
            pltpu.VMEM((1, 2, B * HKV, BLK), jnp.float32),
            pltpu.VMEM((4, B, HKV, BLK, DH), jnp.bfloat16),
            pltpu.VMEM((4, 2, B * HKV, BLK), jnp.float32),
            pltpu.VMEM((2, B, HKV, BLK, DH), jnp.bfloat16),
            pltpu.VMEM((2, 2, B * HKV, BLK), jnp.float32),
            pltpu.VMEM((N_DEV, B * SBLK, D), jnp.bfloat16),
            pltpu.SemaphoreType.DMA((3,)),
            pltpu.SemaphoreType.DMA((3,)),
            pltpu.SemaphoreType.DMA((3,)),
            pltpu.SemaphoreType.DMA((3,)),
            pltpu.SemaphoreType.DMA((7,)),
            pltpu.SemaphoreType.DMA((7,)),
        ],
        compiler_params=pltpu.CompilerParams(collective_id=0),
    )(x, Wq, Wo, K_ext, V_ext)
